# Initial kernel scaffold; baseline (speedup 1.0000x reference)
import jax
import jax.numpy as jnp
from jax.experimental import pallas as pl


def kernel(x, edge_index, W1, b1, A1, A2, W2, b2):
    raise NotImplementedError("write your pallas kernel here")


# speedup vs baseline: 14.6313x; 14.6313x over previous
import functools

import jax
import jax.numpy as jnp
from jax import lax
from jax.experimental import pallas as pl
from jax.experimental.pallas import tpu as pltpu
from jax.experimental.pallas import tpu_sc as plsc

N = 10000
D = 128
HOPS = 10
E = 320000
NCORES = 2
NSUB = 16
NTILES = NCORES * NSUB
EPT = 10240
EPAD = NTILES * EPT
CHUNK = 160
NCHUNK = EPT // CHUNK
NGRP = CHUNK // 16
NPAD = 10112
RPT = NPAD // NSUB
BS = 2000
GRID = N // BS


def _sc_hop_body(h_hbm, x1_hbm, h1_hbm, s_hbm, t_hbm, pn_hbm, pd0_hbm,
                 pd1_hbm, x1t, h1t, sidx, tidx, w1b, rows, dbuf,
                 accn, accd, sem):
    cid = lax.axis_index("c")
    sid = lax.axis_index("s")
    wid = cid * NSUB + sid
    base = sid * RPT

    def _zrow(e, c):
        for d in range(D // 16):
            rows[e, pl.ds(d * 16, 16)] = jnp.zeros((16,), jnp.float32)
        return c
    lax.fori_loop(0, CHUNK, _zrow, 0)
    for g in range(NGRP):
        w1b[pl.ds(g * 16, 16)] = jnp.zeros((16,), jnp.float32)
    for k in range(RPT // CHUNK):
        pltpu.sync_copy(rows, accn.at[pl.ds(base + k * CHUNK, CHUNK)])
        pltpu.sync_copy(w1b, accd.at[pl.ds(base + k * CHUNK, CHUNK)])
    rem = RPT % CHUNK
    ro = base + (RPT // CHUNK) * CHUNK
    pltpu.sync_copy(rows.at[pl.ds(0, rem)], accn.at[pl.ds(ro, rem)])
    pltpu.sync_copy(w1b.at[pl.ds(0, rem)], accd.at[pl.ds(ro, rem)])

    pltpu.sync_copy(x1_hbm, x1t)
    pltpu.sync_copy(h1_hbm, h1t)

    plsc.subcore_barrier()

    def _chunk(ch, c):
        eb = wid * EPT + ch * CHUNK
        pltpu.sync_copy(s_hbm.at[pl.ds(eb, CHUNK)], sidx)
        pltpu.sync_copy(t_hbm.at[pl.ds(eb, CHUNK)], tidx)
        pltpu.async_copy(h_hbm.at[tidx], rows, sem).wait()
        for g in range(NGRP):
            sv = sidx[pl.ds(g * 16, 16)]
            tv = tidx[pl.ds(g * 16, 16)]
            svc = jnp.minimum(sv, N - 1)
            v = (plsc.load_gather(x1t, [svc])
                 + plsc.load_gather(h1t, [tv]))
            v = jnp.where(v > 0, v, 0.2 * v)
            w1b[pl.ds(g * 16, 16)] = jnp.exp(v)

        def _scale(e, c2):
            wb = plsc.load_gather(w1b, [jnp.full((16,), e, jnp.int32)])
            for d in range(D // 16):
                rows[e, pl.ds(d * 16, 16)] = rows[e, pl.ds(d * 16, 16)] * wb
            return c2
        lax.fori_loop(0, CHUNK, _scale, 0)

        pltpu.sync_copy(rows, accn.at[sidx], add=True)
        pltpu.sync_copy(w1b, accd.at[sidx], add=True)
        return c
    lax.fori_loop(0, NCHUNK, _chunk, 0)

    plsc.subcore_barrier()

    def _flush_n(k, c):
        o = base + k * CHUNK
        pltpu.sync_copy(accn.at[pl.ds(o, CHUNK)], rows)
        pltpu.sync_copy(rows, pn_hbm.at[cid, pl.ds(o, CHUNK)])
        return c
    lax.fori_loop(0, RPT // CHUNK, _flush_n, 0)
    pltpu.sync_copy(accn.at[pl.ds(ro, rem)], rows.at[pl.ds(0, rem)])
    pltpu.sync_copy(rows.at[pl.ds(0, rem)], pn_hbm.at[cid, pl.ds(ro, rem)])

    pltpu.sync_copy(accd.at[pl.ds(base, RPT)], dbuf)

    @pl.when(cid == 0)
    def _():
        pltpu.sync_copy(dbuf, pd0_hbm.at[pl.ds(base, RPT)])

    @pl.when(cid == 1)
    def _():
        pltpu.sync_copy(dbuf, pd1_hbm.at[pl.ds(base, RPT)])


_sc_hop = functools.partial(
    pl.kernel,
    out_type=[
        jax.ShapeDtypeStruct((NCORES, NPAD, D), jnp.float32),
        jax.ShapeDtypeStruct((NPAD,), jnp.float32),
        jax.ShapeDtypeStruct((NPAD,), jnp.float32),
    ],
    mesh=plsc.VectorSubcoreMesh(core_axis_name="c", subcore_axis_name="s"),
    compiler_params=pltpu.CompilerParams(needs_layout_passes=False),
    scratch_types=[
        pltpu.VMEM((N,), jnp.float32),
        pltpu.VMEM((N,), jnp.float32),
        pltpu.VMEM((CHUNK,), jnp.int32),
        pltpu.VMEM((CHUNK,), jnp.int32),
        pltpu.VMEM((CHUNK,), jnp.float32),
        pltpu.VMEM((CHUNK, D), jnp.float32),
        pltpu.VMEM((RPT,), jnp.float32),
        pltpu.VMEM_SHARED((NPAD, D), jnp.float32),
        pltpu.VMEM_SHARED((NPAD,), jnp.float32),
        pltpu.SemaphoreType.DMA,
    ],
)(_sc_hop_body)


def _pro_body(x_ref, w1_ref, b1_ref, a1_ref, a2_ref,
              x0_ref, x1_ref, xa2_ref, w2c_ref):
    x0 = lax.dot_general(x_ref[...], w1_ref[...],
                         (((1,), (1,)), ((), ()))) + b1_ref[...]
    x0 = jnp.maximum(x0, 0.0)
    x0_ref[...] = x0
    x1 = lax.dot_general(x0, a1_ref[...], (((1,), (1,)), ((), ())))
    xa2 = lax.dot_general(x0, a2_ref[...], (((1,), (1,)), ((), ())))
    x1_ref[...] = x1
    xa2_ref[...] = xa2
    v = x1 + xa2
    v = jnp.where(v > 0, v, 0.2 * v)
    w2c_ref[...] = jnp.exp(v)


def _prologue(x, w1, b1, a1, a2):
    return pl.pallas_call(
        _pro_body,
        grid=(GRID,),
        in_specs=[
            pl.BlockSpec((BS, D), lambda i: (i, 0)),
            pl.BlockSpec((D, D), lambda i: (0, 0)),
            pl.BlockSpec((1, D), lambda i: (0, 0)),
            pl.BlockSpec((HOPS, D), lambda i: (0, 0)),
            pl.BlockSpec((HOPS, D), lambda i: (0, 0)),
        ],
        out_specs=[
            pl.BlockSpec((BS, D), lambda i: (i, 0)),
            pl.BlockSpec((BS, HOPS), lambda i: (i, 0)),
            pl.BlockSpec((BS, HOPS), lambda i: (i, 0)),
            pl.BlockSpec((BS, HOPS), lambda i: (i, 0)),
        ],
        out_shape=[
            jax.ShapeDtypeStruct((N, D), jnp.float32),
            jax.ShapeDtypeStruct((N, HOPS), jnp.float32),
            jax.ShapeDtypeStruct((N, HOPS), jnp.float32),
            jax.ShapeDtypeStruct((N, HOPS), jnp.float32),
        ],
    )(x, w1, b1, a1, a2)


def _combine(pn, pd, w2, xb):
    num = pn[0] + pn[1] + w2 * xb
    dv = pd[:, 0] + pd[:, 1] + w2[:, 0]
    h = num / dv[:, None]
    return jnp.where(h > 0, h, jnp.exp(h) - 1.0)


def _fin_body(pn_ref, pd_ref, w2_ref, x_ref, a2_ref, h_ref, h1_ref):
    h = _combine(pn_ref[...], pd_ref[...], w2_ref[...], x_ref[...])
    h_ref[...] = h
    h1_ref[...] = jnp.dot(h, a2_ref[...][0])[:, None]


def _finalize(pn, pd, w2, x0, a2n):
    return pl.pallas_call(
        _fin_body,
        grid=(GRID,),
        in_specs=[
            pl.BlockSpec((NCORES, BS, D), lambda i: (0, i, 0)),
            pl.BlockSpec((BS, NCORES), lambda i: (i, 0)),
            pl.BlockSpec((BS, 1), lambda i: (i, 0)),
            pl.BlockSpec((BS, D), lambda i: (i, 0)),
            pl.BlockSpec((1, D), lambda i: (0, 0)),
        ],
        out_specs=[
            pl.BlockSpec((BS, D), lambda i: (i, 0)),
            pl.BlockSpec((BS, 1), lambda i: (i, 0)),
        ],
        out_shape=[
            jax.ShapeDtypeStruct((N, D), jnp.float32),
            jax.ShapeDtypeStruct((N, 1), jnp.float32),
        ],
    )(pn, pd, w2, x0, a2n)


def _fin_last_body(pn_ref, pd_ref, w2_ref, x_ref, w2m_ref, b2_ref, o_ref):
    h = _combine(pn_ref[...], pd_ref[...], w2_ref[...], x_ref[...])
    o_ref[...] = lax.dot_general(h, w2m_ref[...],
                                 (((1,), (1,)), ((), ()))) + b2_ref[...]


def _finalize_last(pn, pd, w2, x0, w2m, b2):
    return pl.pallas_call(
        _fin_last_body,
        grid=(GRID,),
        in_specs=[
            pl.BlockSpec((NCORES, BS, D), lambda i: (0, i, 0)),
            pl.BlockSpec((BS, NCORES), lambda i: (i, 0)),
            pl.BlockSpec((BS, 1), lambda i: (i, 0)),
            pl.BlockSpec((BS, D), lambda i: (i, 0)),
            pl.BlockSpec((D, D), lambda i: (0, 0)),
            pl.BlockSpec((1, D), lambda i: (0, 0)),
        ],
        out_specs=pl.BlockSpec((BS, D), lambda i: (i, 0)),
        out_shape=jax.ShapeDtypeStruct((N, D), jnp.float32),
    )(pn, pd, w2, x0, w2m, b2)


def kernel(x, edge_index, W1, b1, A1, A2, W2, b2):
    s = edge_index[0].astype(jnp.int32)
    t = edge_index[1].astype(jnp.int32)
    npad = EPAD - E
    ar = jnp.arange(npad, dtype=jnp.int32)
    sarr = jnp.concatenate([s, N + (ar % (NPAD - N))])
    tarr = jnp.concatenate([t, ar % N])

    x0, x1a, xa2, w2c = _prologue(x, W1, b1.reshape(1, D), A1, A2)
    h = x0
    h1 = xa2[:, 0]
    out = None
    for i in range(HOPS):
        pn, pd0, pd1 = _sc_hop(h, x1a[:, i], h1, sarr, tarr)
        pdt = jnp.stack([pd0, pd1], axis=1)
        w2r = w2c[:, i].reshape(N, 1)
        if i < HOPS - 1:
            h, h1r = _finalize(pn, pdt, w2r, x0, A2[i + 1].reshape(1, D))
            h1 = h1r[:, 0]
        else:
            out = _finalize_last(pn, pdt, w2r, x0, W2, b2.reshape(1, D))
    return out


# speedup vs baseline: 20.7796x; 1.4202x over previous
import functools

import jax
import jax.numpy as jnp
from jax import lax
from jax.experimental import pallas as pl
from jax.experimental.pallas import tpu as pltpu
from jax.experimental.pallas import tpu_sc as plsc

N = 10000
D = 128
HOPS = 10
E = 320000
NCORES = 2
NSUB = 16
NTILES = NCORES * NSUB
EPT = 10240
EPAD = NTILES * EPT
CHUNK = 160
NCHUNK = EPT // CHUNK
NGRP = CHUNK // 16
NITER = NCHUNK // 2
NPAD = 10112
RPT = NPAD // NSUB
BS = 2000
GRID = N // BS


def _edge_weights(x1v, h1v, w1b):
    for g in range(NGRP):
        v = x1v[pl.ds(g * 16, 16)] + h1v[pl.ds(g * 16, 16)]
        v = jnp.where(v > 0, v, 0.2 * v)
        w1b[pl.ds(g * 16, 16)] = jnp.exp(v)


def _scale_scatter(sidx, w1b, rows, accn, accd):
    def _scale(g, c2):
        for j in range(16):
            e = g * 16 + j
            wb = plsc.load_gather(w1b, [jnp.full((16,), e, jnp.int32)])
            for d in range(D // 16):
                rows[e, pl.ds(d * 16, 16)] = rows[e, pl.ds(d * 16, 16)] * wb
        return c2
    lax.fori_loop(0, NGRP, _scale, 0)
    pltpu.sync_copy(rows, accn.at[sidx], add=True)
    pltpu.sync_copy(w1b, accd.at[sidx], add=True)


def _stage(ea, s_hbm, t_hbm, h_hbm, x1_hbm, h1_hbm,
           sidx, tidx, x1v, h1v, rows, semr, seme):
    pltpu.sync_copy(s_hbm.at[pl.ds(ea, CHUNK)], sidx)
    pltpu.sync_copy(t_hbm.at[pl.ds(ea, CHUNK)], tidx)
    pltpu.async_copy(h_hbm.at[tidx], rows, semr)
    pltpu.async_copy(x1_hbm.at[sidx], x1v, seme)
    pltpu.async_copy(h1_hbm.at[tidx], h1v, seme)


def _process(h_hbm, x1_hbm, h1_hbm, sidx, tidx, x1v, h1v, w1b, rows,
             semr, seme, accn, accd):
    pltpu.make_async_copy(x1_hbm.at[sidx], x1v, seme).wait()
    pltpu.make_async_copy(h1_hbm.at[tidx], h1v, seme).wait()
    _edge_weights(x1v, h1v, w1b)
    pltpu.make_async_copy(h_hbm.at[tidx], rows, semr).wait()
    _scale_scatter(sidx, w1b, rows, accn, accd)


def _sc_hop_body(h_hbm, x1_hbm, h1_hbm, s_hbm, t_hbm, pn_hbm, pd0_hbm,
                 pd1_hbm, sidx0, tidx0, sidx1, tidx1, x1v0, h1v0, x1v1,
                 h1v1, w1b0, w1b1, rows0, rows1, dbuf, accn, accd,
                 semr0, seme0, semr1, seme1):
    cid = lax.axis_index("c")
    sid = lax.axis_index("s")
    wid = cid * NSUB + sid
    base = sid * RPT

    def _zrow(e, c):
        for d in range(D // 16):
            rows0[e, pl.ds(d * 16, 16)] = jnp.zeros((16,), jnp.float32)
        return c
    lax.fori_loop(0, CHUNK, _zrow, 0)
    for g in range(NGRP):
        w1b0[pl.ds(g * 16, 16)] = jnp.zeros((16,), jnp.float32)
    rem = RPT % CHUNK
    ro = base + (RPT // CHUNK) * CHUNK
    for k in range(RPT // CHUNK):
        pltpu.sync_copy(rows0, accn.at[pl.ds(base + k * CHUNK, CHUNK)])
        pltpu.sync_copy(w1b0, accd.at[pl.ds(base + k * CHUNK, CHUNK)])
    pltpu.sync_copy(rows0.at[pl.ds(0, rem)], accn.at[pl.ds(ro, rem)])
    pltpu.sync_copy(w1b0.at[pl.ds(0, rem)], accd.at[pl.ds(ro, rem)])

    plsc.subcore_barrier()

    eb0 = wid * EPT
    _stage(eb0, s_hbm, t_hbm, h_hbm, x1_hbm, h1_hbm,
           sidx0, tidx0, x1v0, h1v0, rows0, semr0, seme0)
    _stage(eb0 + CHUNK, s_hbm, t_hbm, h_hbm, x1_hbm, h1_hbm,
           sidx1, tidx1, x1v1, h1v1, rows1, semr1, seme1)

    def _iter(k, c):
        ea = eb0 + 2 * k * CHUNK
        _process(h_hbm, x1_hbm, h1_hbm, sidx0, tidx0, x1v0, h1v0, w1b0,
                 rows0, semr0, seme0, accn, accd)

        @pl.when(k < NITER - 1)
        def _():
            _stage(ea + 2 * CHUNK, s_hbm, t_hbm, h_hbm, x1_hbm, h1_hbm,
                   sidx0, tidx0, x1v0, h1v0, rows0, semr0, seme0)

        _process(h_hbm, x1_hbm, h1_hbm, sidx1, tidx1, x1v1, h1v1, w1b1,
                 rows1, semr1, seme1, accn, accd)

        @pl.when(k < NITER - 1)
        def _():
            _stage(ea + 3 * CHUNK, s_hbm, t_hbm, h_hbm, x1_hbm, h1_hbm,
                   sidx1, tidx1, x1v1, h1v1, rows1, semr1, seme1)
        return c
    lax.fori_loop(0, NITER, _iter, 0)

    plsc.subcore_barrier()

    for k in range(RPT // CHUNK):
        buf = rows0 if k % 2 == 0 else rows1
        pltpu.sync_copy(accn.at[pl.ds(base + k * CHUNK, CHUNK)], buf)
        pltpu.sync_copy(buf, pn_hbm.at[cid, pl.ds(base + k * CHUNK, CHUNK)])
    pltpu.sync_copy(accn.at[pl.ds(ro, rem)], rows0.at[pl.ds(0, rem)])
    pltpu.sync_copy(rows0.at[pl.ds(0, rem)], pn_hbm.at[cid, pl.ds(ro, rem)])

    pltpu.sync_copy(accd.at[pl.ds(base, RPT)], dbuf)

    @pl.when(cid == 0)
    def _():
        pltpu.sync_copy(dbuf, pd0_hbm.at[pl.ds(base, RPT)])

    @pl.when(cid == 1)
    def _():
        pltpu.sync_copy(dbuf, pd1_hbm.at[pl.ds(base, RPT)])


_sc_hop = functools.partial(
    pl.kernel,
    out_type=[
        jax.ShapeDtypeStruct((NCORES, NPAD, D), jnp.float32),
        jax.ShapeDtypeStruct((NPAD,), jnp.float32),
        jax.ShapeDtypeStruct((NPAD,), jnp.float32),
    ],
    mesh=plsc.VectorSubcoreMesh(core_axis_name="c", subcore_axis_name="s"),
    compiler_params=pltpu.CompilerParams(needs_layout_passes=False),
    scratch_types=[
        pltpu.VMEM((CHUNK,), jnp.int32),
        pltpu.VMEM((CHUNK,), jnp.int32),
        pltpu.VMEM((CHUNK,), jnp.int32),
        pltpu.VMEM((CHUNK,), jnp.int32),
        pltpu.VMEM((CHUNK,), jnp.float32),
        pltpu.VMEM((CHUNK,), jnp.float32),
        pltpu.VMEM((CHUNK,), jnp.float32),
        pltpu.VMEM((CHUNK,), jnp.float32),
        pltpu.VMEM((CHUNK,), jnp.float32),
        pltpu.VMEM((CHUNK,), jnp.float32),
        pltpu.VMEM((CHUNK, D), jnp.float32),
        pltpu.VMEM((CHUNK, D), jnp.float32),
        pltpu.VMEM((RPT,), jnp.float32),
        pltpu.VMEM_SHARED((NPAD, D), jnp.float32),
        pltpu.VMEM_SHARED((NPAD,), jnp.float32),
        pltpu.SemaphoreType.DMA,
        pltpu.SemaphoreType.DMA,
        pltpu.SemaphoreType.DMA,
        pltpu.SemaphoreType.DMA,
    ],
)(_sc_hop_body)


def _pro_body(x_ref, w1_ref, b1_ref, a1_ref, a2_ref,
              x0_ref, x1_ref, xa2_ref, w2c_ref):
    x0 = lax.dot_general(x_ref[...], w1_ref[...],
                         (((1,), (1,)), ((), ()))) + b1_ref[...]
    x0 = jnp.maximum(x0, 0.0)
    x0_ref[...] = x0
    x1 = lax.dot_general(x0, a1_ref[...], (((1,), (1,)), ((), ())))
    xa2 = lax.dot_general(x0, a2_ref[...], (((1,), (1,)), ((), ())))
    x1_ref[...] = x1
    xa2_ref[...] = xa2
    v = x1 + xa2
    v = jnp.where(v > 0, v, 0.2 * v)
    w2c_ref[...] = jnp.exp(v)


def _prologue(x, w1, b1, a1, a2):
    return pl.pallas_call(
        _pro_body,
        grid=(GRID,),
        in_specs=[
            pl.BlockSpec((BS, D), lambda i: (i, 0)),
            pl.BlockSpec((D, D), lambda i: (0, 0)),
            pl.BlockSpec((1, D), lambda i: (0, 0)),
            pl.BlockSpec((HOPS, D), lambda i: (0, 0)),
            pl.BlockSpec((HOPS, D), lambda i: (0, 0)),
        ],
        out_specs=[
            pl.BlockSpec((BS, D), lambda i: (i, 0)),
            pl.BlockSpec((BS, HOPS), lambda i: (i, 0)),
            pl.BlockSpec((BS, HOPS), lambda i: (i, 0)),
            pl.BlockSpec((BS, HOPS), lambda i: (i, 0)),
        ],
        out_shape=[
            jax.ShapeDtypeStruct((N, D), jnp.float32),
            jax.ShapeDtypeStruct((N, HOPS), jnp.float32),
            jax.ShapeDtypeStruct((N, HOPS), jnp.float32),
            jax.ShapeDtypeStruct((N, HOPS), jnp.float32),
        ],
    )(x, w1, b1, a1, a2)


def _combine(pn, pd, w2, xb):
    num = pn[0] + pn[1] + w2 * xb
    dv = pd[:, 0] + pd[:, 1] + w2[:, 0]
    h = num / dv[:, None]
    return jnp.where(h > 0, h, jnp.exp(h) - 1.0)


def _fin_body(pn_ref, pd_ref, w2_ref, x_ref, a2_ref, h_ref, h1_ref):
    h = _combine(pn_ref[...], pd_ref[...], w2_ref[...], x_ref[...])
    h_ref[...] = h
    h1_ref[...] = jnp.dot(h, a2_ref[...][0])[:, None]


def _finalize(pn, pd, w2, x0, a2n):
    return pl.pallas_call(
        _fin_body,
        grid=(GRID,),
        in_specs=[
            pl.BlockSpec((NCORES, BS, D), lambda i: (0, i, 0)),
            pl.BlockSpec((BS, NCORES), lambda i: (i, 0)),
            pl.BlockSpec((BS, 1), lambda i: (i, 0)),
            pl.BlockSpec((BS, D), lambda i: (i, 0)),
            pl.BlockSpec((1, D), lambda i: (0, 0)),
        ],
        out_specs=[
            pl.BlockSpec((BS, D), lambda i: (i, 0)),
            pl.BlockSpec((BS, 1), lambda i: (i, 0)),
        ],
        out_shape=[
            jax.ShapeDtypeStruct((N, D), jnp.float32),
            jax.ShapeDtypeStruct((N, 1), jnp.float32),
        ],
    )(pn, pd, w2, x0, a2n)


def _fin_last_body(pn_ref, pd_ref, w2_ref, x_ref, w2m_ref, b2_ref, o_ref):
    h = _combine(pn_ref[...], pd_ref[...], w2_ref[...], x_ref[...])
    o_ref[...] = lax.dot_general(h, w2m_ref[...],
                                 (((1,), (1,)), ((), ()))) + b2_ref[...]


def _finalize_last(pn, pd, w2, x0, w2m, b2):
    return pl.pallas_call(
        _fin_last_body,
        grid=(GRID,),
        in_specs=[
            pl.BlockSpec((NCORES, BS, D), lambda i: (0, i, 0)),
            pl.BlockSpec((BS, NCORES), lambda i: (i, 0)),
            pl.BlockSpec((BS, 1), lambda i: (i, 0)),
            pl.BlockSpec((BS, D), lambda i: (i, 0)),
            pl.BlockSpec((D, D), lambda i: (0, 0)),
            pl.BlockSpec((1, D), lambda i: (0, 0)),
        ],
        out_specs=pl.BlockSpec((BS, D), lambda i: (i, 0)),
        out_shape=jax.ShapeDtypeStruct((N, D), jnp.float32),
    )(pn, pd, w2, x0, w2m, b2)


def kernel(x, edge_index, W1, b1, A1, A2, W2, b2):
    s = edge_index[0].astype(jnp.int32)
    t = edge_index[1].astype(jnp.int32)
    npad = EPAD - E
    ar = jnp.arange(npad, dtype=jnp.int32)
    sarr = jnp.concatenate([s, N + (ar % (NPAD - N))])
    tarr = jnp.concatenate([t, ar % N])

    x0, x1a, xa2, w2c = _prologue(x, W1, b1.reshape(1, D), A1, A2)
    zpad = jnp.zeros((NPAD - N,), jnp.float32)
    h = x0
    h1 = xa2[:, 0]
    out = None
    for i in range(HOPS):
        x1p = jnp.concatenate([x1a[:, i], zpad])
        h1p = jnp.concatenate([h1, zpad])
        pn, pd0, pd1 = _sc_hop(h, x1p, h1p, sarr, tarr)
        pdt = jnp.stack([pd0, pd1], axis=1)
        w2r = w2c[:, i].reshape(N, 1)
        if i < HOPS - 1:
            h, h1r = _finalize(pn, pdt, w2r, x0, A2[i + 1].reshape(1, D))
            h1 = h1r[:, 0]
        else:
            out = _finalize_last(pn, pdt, w2r, x0, W2, b2.reshape(1, D))
    return out


# speedup vs baseline: 23.3392x; 1.1232x over previous
import functools

import jax
import jax.numpy as jnp
from jax import lax
from jax.experimental import pallas as pl
from jax.experimental.pallas import tpu as pltpu
from jax.experimental.pallas import tpu_sc as plsc

N = 10000
D = 128
HOPS = 10
E = 320000
NCORES = 2
NSUB = 16
NTILES = NCORES * NSUB
EPT = 10240
EPAD = NTILES * EPT
CHUNK = 160
NCHUNK = EPT // CHUNK
NGRP = CHUNK // 16
NITER = NCHUNK // 2
NPAD = 10112
RPT = NPAD // NSUB
BS = 2000
GRID = N // BS


def _edge_weights(x1v, h1v, w1b):
    for g in range(NGRP):
        v = x1v[pl.ds(g * 16, 16)] + h1v[pl.ds(g * 16, 16)]
        v = jnp.where(v > 0, v, 0.2 * v)
        w1b[pl.ds(g * 16, 16)] = jnp.exp(v)


def _scale_scatter(sidx, w1b, rows, accn, accd, sems):
    def _scale(g, c2):
        for j in range(16):
            e = g * 16 + j
            wb = plsc.load_gather(w1b, [jnp.full((16,), e, jnp.int32)])
            for d in range(D // 16):
                rows[e, pl.ds(d * 16, 16)] = rows[e, pl.ds(d * 16, 16)] * wb
        return c2
    lax.fori_loop(0, NGRP, _scale, 0)
    pltpu.sync_copy(rows, accn.at[sidx], add=True)
    pltpu.async_copy(w1b, accd.at[sidx], sems, add=True)


def _stage(ch, st_hbm, h_hbm, x1_hbm, h1_hbm,
           stb, sidx, x1v, h1v, rows, semr, seme):
    pltpu.sync_copy(st_hbm.at[pl.ds(ch * 2 * CHUNK, 2 * CHUNK)], stb)
    tref = stb.at[pl.ds(CHUNK, CHUNK)]
    sref = stb.at[pl.ds(0, CHUNK)]
    pltpu.async_copy(h_hbm.at[tref], rows, semr)
    pltpu.async_copy(x1_hbm.at[sref], x1v, seme)
    pltpu.async_copy(h1_hbm.at[tref], h1v, seme)


def _process(h_hbm, x1_hbm, h1_hbm, stb, sidx, x1v, h1v, w1b, rows,
             semr, seme, sems, accn, accd, first):
    @pl.when(jnp.logical_not(first))
    def _():
        pltpu.make_async_copy(h1_hbm.at[sidx], w1b, sems).wait()
    for g in range(NGRP):
        sidx[pl.ds(g * 16, 16)] = stb[pl.ds(g * 16, 16)]
    sref = stb.at[pl.ds(0, CHUNK)]
    tref = stb.at[pl.ds(CHUNK, CHUNK)]
    pltpu.make_async_copy(x1_hbm.at[sref], x1v, seme).wait()
    pltpu.make_async_copy(h1_hbm.at[tref], h1v, seme).wait()
    _edge_weights(x1v, h1v, w1b)
    pltpu.make_async_copy(h_hbm.at[tref], rows, semr).wait()
    _scale_scatter(sidx, w1b, rows, accn, accd, sems)


def _sc_hop_body(h_hbm, x1_hbm, h1_hbm, st_hbm, pn_hbm, pd0_hbm,
                 pd1_hbm, stb0, sidx0, stb1, sidx1, x1v0, h1v0, x1v1,
                 h1v1, w1b0, w1b1, rows0, rows1, dbuf, accn, accd,
                 semr0, seme0, sems0, semr1, seme1, sems1):
    cid = lax.axis_index("c")
    sid = lax.axis_index("s")
    wid = cid * NSUB + sid
    base = sid * RPT

    def _zrow(e, c):
        for d in range(D // 16):
            rows0[e, pl.ds(d * 16, 16)] = jnp.zeros((16,), jnp.float32)
        return c
    lax.fori_loop(0, CHUNK, _zrow, 0)
    for g in range(NGRP):
        w1b0[pl.ds(g * 16, 16)] = jnp.zeros((16,), jnp.float32)
    rem = RPT % CHUNK
    ro = base + (RPT // CHUNK) * CHUNK
    for k in range(RPT // CHUNK):
        pltpu.sync_copy(rows0, accn.at[pl.ds(base + k * CHUNK, CHUNK)])
        pltpu.sync_copy(w1b0, accd.at[pl.ds(base + k * CHUNK, CHUNK)])
    pltpu.sync_copy(rows0.at[pl.ds(0, rem)], accn.at[pl.ds(ro, rem)])
    pltpu.sync_copy(w1b0.at[pl.ds(0, rem)], accd.at[pl.ds(ro, rem)])

    plsc.subcore_barrier()

    cb0 = wid * NCHUNK
    _stage(cb0, st_hbm, h_hbm, x1_hbm, h1_hbm,
           stb0, sidx0, x1v0, h1v0, rows0, semr0, seme0)
    _stage(cb0 + 1, st_hbm, h_hbm, x1_hbm, h1_hbm,
           stb1, sidx1, x1v1, h1v1, rows1, semr1, seme1)

    def _iter(k, c):
        ca = cb0 + 2 * k
        _process(h_hbm, x1_hbm, h1_hbm, stb0, sidx0, x1v0, h1v0, w1b0,
                 rows0, semr0, seme0, sems0, accn, accd, k == 0)

        @pl.when(k < NITER - 1)
        def _():
            _stage(ca + 2, st_hbm, h_hbm, x1_hbm, h1_hbm,
                   stb0, sidx0, x1v0, h1v0, rows0, semr0, seme0)

        _process(h_hbm, x1_hbm, h1_hbm, stb1, sidx1, x1v1, h1v1, w1b1,
                 rows1, semr1, seme1, sems1, accn, accd, k == 0)

        @pl.when(k < NITER - 1)
        def _():
            _stage(ca + 3, st_hbm, h_hbm, x1_hbm, h1_hbm,
                   stb1, sidx1, x1v1, h1v1, rows1, semr1, seme1)
        return c
    lax.fori_loop(0, NITER, _iter, 0)

    pltpu.make_async_copy(h1_hbm.at[sidx0], w1b0, sems0).wait()
    pltpu.make_async_copy(h1_hbm.at[sidx1], w1b1, sems1).wait()

    plsc.subcore_barrier()

    for k in range(RPT // CHUNK):
        buf = rows0 if k % 2 == 0 else rows1
        pltpu.sync_copy(accn.at[pl.ds(base + k * CHUNK, CHUNK)], buf)
        pltpu.sync_copy(buf, pn_hbm.at[cid, pl.ds(base + k * CHUNK, CHUNK)])
    pltpu.sync_copy(accn.at[pl.ds(ro, rem)], rows0.at[pl.ds(0, rem)])
    pltpu.sync_copy(rows0.at[pl.ds(0, rem)], pn_hbm.at[cid, pl.ds(ro, rem)])

    pltpu.sync_copy(accd.at[pl.ds(base, RPT)], dbuf)

    @pl.when(cid == 0)
    def _():
        pltpu.sync_copy(dbuf, pd0_hbm.at[pl.ds(base, RPT)])

    @pl.when(cid == 1)
    def _():
        pltpu.sync_copy(dbuf, pd1_hbm.at[pl.ds(base, RPT)])


_sc_hop = functools.partial(
    pl.kernel,
    out_type=[
        jax.ShapeDtypeStruct((NCORES, NPAD, D), jnp.float32),
        jax.ShapeDtypeStruct((NPAD,), jnp.float32),
        jax.ShapeDtypeStruct((NPAD,), jnp.float32),
    ],
    mesh=plsc.VectorSubcoreMesh(core_axis_name="c", subcore_axis_name="s"),
    compiler_params=pltpu.CompilerParams(needs_layout_passes=False),
    scratch_types=[
        pltpu.VMEM((2 * CHUNK,), jnp.int32),
        pltpu.VMEM((CHUNK,), jnp.int32),
        pltpu.VMEM((2 * CHUNK,), jnp.int32),
        pltpu.VMEM((CHUNK,), jnp.int32),
        pltpu.VMEM((CHUNK,), jnp.float32),
        pltpu.VMEM((CHUNK,), jnp.float32),
        pltpu.VMEM((CHUNK,), jnp.float32),
        pltpu.VMEM((CHUNK,), jnp.float32),
        pltpu.VMEM((CHUNK,), jnp.float32),
        pltpu.VMEM((CHUNK,), jnp.float32),
        pltpu.VMEM((CHUNK, D), jnp.float32),
        pltpu.VMEM((CHUNK, D), jnp.float32),
        pltpu.VMEM((RPT,), jnp.float32),
        pltpu.VMEM_SHARED((NPAD, D), jnp.float32),
        pltpu.VMEM_SHARED((NPAD,), jnp.float32),
        pltpu.SemaphoreType.DMA,
        pltpu.SemaphoreType.DMA,
        pltpu.SemaphoreType.DMA,
        pltpu.SemaphoreType.DMA,
        pltpu.SemaphoreType.DMA,
        pltpu.SemaphoreType.DMA,
    ],
)(_sc_hop_body)


def _pro_body(x_ref, w1_ref, b1_ref, a1_ref, a2_ref,
              x0_ref, x1_ref, xa2_ref, w2c_ref):
    x0 = lax.dot_general(x_ref[...], w1_ref[...],
                         (((1,), (1,)), ((), ()))) + b1_ref[...]
    x0 = jnp.maximum(x0, 0.0)
    x0_ref[...] = x0
    x1 = lax.dot_general(x0, a1_ref[...], (((1,), (1,)), ((), ())))
    xa2 = lax.dot_general(x0, a2_ref[...], (((1,), (1,)), ((), ())))
    x1_ref[...] = x1
    xa2_ref[...] = xa2
    v = x1 + xa2
    v = jnp.where(v > 0, v, 0.2 * v)
    w2c_ref[...] = jnp.exp(v)


def _prologue(x, w1, b1, a1, a2):
    return pl.pallas_call(
        _pro_body,
        grid=(GRID,),
        in_specs=[
            pl.BlockSpec((BS, D), lambda i: (i, 0)),
            pl.BlockSpec((D, D), lambda i: (0, 0)),
            pl.BlockSpec((1, D), lambda i: (0, 0)),
            pl.BlockSpec((HOPS, D), lambda i: (0, 0)),
            pl.BlockSpec((HOPS, D), lambda i: (0, 0)),
        ],
        out_specs=[
            pl.BlockSpec((BS, D), lambda i: (i, 0)),
            pl.BlockSpec((BS, HOPS), lambda i: (i, 0)),
            pl.BlockSpec((BS, HOPS), lambda i: (i, 0)),
            pl.BlockSpec((BS, HOPS), lambda i: (i, 0)),
        ],
        out_shape=[
            jax.ShapeDtypeStruct((N, D), jnp.float32),
            jax.ShapeDtypeStruct((N, HOPS), jnp.float32),
            jax.ShapeDtypeStruct((N, HOPS), jnp.float32),
            jax.ShapeDtypeStruct((N, HOPS), jnp.float32),
        ],
    )(x, w1, b1, a1, a2)


def _combine(pn, pd, w2, xb):
    num = pn[0] + pn[1] + w2 * xb
    dv = pd[:, 0] + pd[:, 1] + w2[:, 0]
    h = num / dv[:, None]
    return jnp.where(h > 0, h, jnp.exp(h) - 1.0)


def _fin_body(pn_ref, pd_ref, w2_ref, x_ref, a2_ref, h_ref, h1_ref):
    h = _combine(pn_ref[...], pd_ref[...], w2_ref[...], x_ref[...])
    h_ref[...] = h
    h1_ref[...] = jnp.dot(h, a2_ref[...][0])[:, None]


def _finalize(pn, pd, w2, x0, a2n):
    return pl.pallas_call(
        _fin_body,
        grid=(GRID,),
        in_specs=[
            pl.BlockSpec((NCORES, BS, D), lambda i: (0, i, 0)),
            pl.BlockSpec((BS, NCORES), lambda i: (i, 0)),
            pl.BlockSpec((BS, 1), lambda i: (i, 0)),
            pl.BlockSpec((BS, D), lambda i: (i, 0)),
            pl.BlockSpec((1, D), lambda i: (0, 0)),
        ],
        out_specs=[
            pl.BlockSpec((BS, D), lambda i: (i, 0)),
            pl.BlockSpec((BS, 1), lambda i: (i, 0)),
        ],
        out_shape=[
            jax.ShapeDtypeStruct((N, D), jnp.float32),
            jax.ShapeDtypeStruct((N, 1), jnp.float32),
        ],
    )(pn, pd, w2, x0, a2n)


def _fin_last_body(pn_ref, pd_ref, w2_ref, x_ref, w2m_ref, b2_ref, o_ref):
    h = _combine(pn_ref[...], pd_ref[...], w2_ref[...], x_ref[...])
    o_ref[...] = lax.dot_general(h, w2m_ref[...],
                                 (((1,), (1,)), ((), ()))) + b2_ref[...]


def _finalize_last(pn, pd, w2, x0, w2m, b2):
    return pl.pallas_call(
        _fin_last_body,
        grid=(GRID,),
        in_specs=[
            pl.BlockSpec((NCORES, BS, D), lambda i: (0, i, 0)),
            pl.BlockSpec((BS, NCORES), lambda i: (i, 0)),
            pl.BlockSpec((BS, 1), lambda i: (i, 0)),
            pl.BlockSpec((BS, D), lambda i: (i, 0)),
            pl.BlockSpec((D, D), lambda i: (0, 0)),
            pl.BlockSpec((1, D), lambda i: (0, 0)),
        ],
        out_specs=pl.BlockSpec((BS, D), lambda i: (i, 0)),
        out_shape=jax.ShapeDtypeStruct((N, D), jnp.float32),
    )(pn, pd, w2, x0, w2m, b2)


def kernel(x, edge_index, W1, b1, A1, A2, W2, b2):
    s = edge_index[0].astype(jnp.int32)
    t = edge_index[1].astype(jnp.int32)
    npad = EPAD - E
    ar = jnp.arange(npad, dtype=jnp.int32)
    sarr = jnp.concatenate([s, N + (ar % (NPAD - N))])
    tarr = jnp.concatenate([t, ar % N])
    stline = jnp.stack([sarr.reshape(-1, CHUNK),
                        tarr.reshape(-1, CHUNK)], axis=1).reshape(-1)

    x0, x1a, xa2, w2c = _prologue(x, W1, b1.reshape(1, D), A1, A2)
    zpad = jnp.zeros((NPAD - N,), jnp.float32)
    h = x0
    h1 = xa2[:, 0]
    out = None
    for i in range(HOPS):
        x1p = jnp.concatenate([x1a[:, i], zpad])
        h1p = jnp.concatenate([h1, zpad])
        pn, pd0, pd1 = _sc_hop(h, x1p, h1p, stline)
        pdt = jnp.stack([pd0, pd1], axis=1)
        w2r = w2c[:, i].reshape(N, 1)
        if i < HOPS - 1:
            h, h1r = _finalize(pn, pdt, w2r, x0, A2[i + 1].reshape(1, D))
            h1 = h1r[:, 0]
        else:
            out = _finalize_last(pn, pdt, w2r, x0, W2, b2.reshape(1, D))
    return out


# speedup vs baseline: 26.9604x; 1.1552x over previous
import functools

import jax
import jax.numpy as jnp
from jax import lax
from jax.experimental import pallas as pl
from jax.experimental.pallas import tpu as pltpu
from jax.experimental.pallas import tpu_sc as plsc

N = 10000
D = 128
HOPS = 10
E = 320000
NCORES = 2
NSUB = 16
NTILES = NCORES * NSUB
CHUNK = 112
NCHUNK = 93
EPT = CHUNK * NCHUNK
EPAD = NTILES * EPT
NGRP = CHUNK // 16
NTRI = NCHUNK // 3
NPAD = 10112
RPT = NPAD // NSUB
BS = 2000
GRID = N // BS


def _edge_weights(x1v, h1v, w1b):
    for g in range(NGRP):
        v = x1v[pl.ds(g * 16, 16)] + h1v[pl.ds(g * 16, 16)]
        v = jnp.where(v > 0, v, 0.2 * v)
        w1b[pl.ds(g * 16, 16)] = jnp.exp(v)


def _scale_scatter(sidx, w1b, rows, accn, accd, sems):
    def _scale(g, c2):
        for j in range(16):
            e = g * 16 + j
            wb = plsc.load_gather(w1b, [jnp.full((16,), e, jnp.int32)])
            for d in range(D // 16):
                rows[e, pl.ds(d * 16, 16)] = rows[e, pl.ds(d * 16, 16)] * wb
        return c2
    lax.fori_loop(0, NGRP, _scale, 0)
    pltpu.async_copy(rows, accn.at[sidx], sems, add=True)
    pltpu.async_copy(w1b, accd.at[sidx], sems, add=True)


def _stage(ch, st_hbm, h_hbm, x1_hbm, h1_hbm,
           stb, sidx, x1v, h1v, rows, semr, seme):
    pltpu.sync_copy(st_hbm.at[pl.ds(ch * 2 * CHUNK, 2 * CHUNK)], stb)
    tref = stb.at[pl.ds(CHUNK, CHUNK)]
    sref = stb.at[pl.ds(0, CHUNK)]
    pltpu.async_copy(h_hbm.at[tref], rows, semr)
    pltpu.async_copy(x1_hbm.at[sref], x1v, seme)
    pltpu.async_copy(h1_hbm.at[tref], h1v, seme)


def _drain_scatters(h_hbm, h1_hbm, stb, sidx, w1b, rows, sems):
    tref = stb.at[pl.ds(CHUNK, CHUNK)]
    pltpu.make_async_copy(h_hbm.at[tref], rows, sems).wait()
    pltpu.make_async_copy(h1_hbm.at[sidx], w1b, sems).wait()


def _process(h_hbm, x1_hbm, h1_hbm, stb, sidx, x1v, h1v, w1b, rows,
             semr, seme, sems, accn, accd):
    for g in range(NGRP):
        sidx[pl.ds(g * 16, 16)] = stb[pl.ds(g * 16, 16)]
    sref = stb.at[pl.ds(0, CHUNK)]
    tref = stb.at[pl.ds(CHUNK, CHUNK)]
    pltpu.make_async_copy(x1_hbm.at[sref], x1v, seme).wait()
    pltpu.make_async_copy(h1_hbm.at[tref], h1v, seme).wait()
    _edge_weights(x1v, h1v, w1b)
    pltpu.make_async_copy(h_hbm.at[tref], rows, semr).wait()
    _scale_scatter(sidx, w1b, rows, accn, accd, sems)


def _sc_hop_body(h_hbm, x1_hbm, h1_hbm, st_hbm, pn_hbm, pd0_hbm,
                 pd1_hbm, stb0, sidx0, stb1, sidx1, stb2, sidx2,
                 x1v0, h1v0, x1v1, h1v1, x1v2, h1v2, w1b0, w1b1, w1b2,
                 rows0, rows1, rows2, dbuf, accn, accd,
                 semr0, seme0, sems0, semr1, seme1, sems1,
                 semr2, seme2, sems2):
    cid = lax.axis_index("c")
    sid = lax.axis_index("s")
    wid = cid * NSUB + sid
    base = sid * RPT

    def _zrow(e, c):
        for d in range(D // 16):
            rows0[e, pl.ds(d * 16, 16)] = jnp.zeros((16,), jnp.float32)
        return c
    lax.fori_loop(0, CHUNK, _zrow, 0)
    for g in range(NGRP):
        w1b0[pl.ds(g * 16, 16)] = jnp.zeros((16,), jnp.float32)
    rem = RPT % CHUNK
    ro = base + (RPT // CHUNK) * CHUNK
    for k in range(RPT // CHUNK):
        pltpu.sync_copy(rows0, accn.at[pl.ds(base + k * CHUNK, CHUNK)])
        pltpu.sync_copy(w1b0, accd.at[pl.ds(base + k * CHUNK, CHUNK)])
    pltpu.sync_copy(rows0.at[pl.ds(0, rem)], accn.at[pl.ds(ro, rem)])
    pltpu.sync_copy(w1b0.at[pl.ds(0, rem)], accd.at[pl.ds(ro, rem)])

    plsc.subcore_barrier()

    cb0 = wid * NCHUNK
    bufs = ((stb0, sidx0, x1v0, h1v0, w1b0, rows0, semr0, seme0, sems0),
            (stb1, sidx1, x1v1, h1v1, w1b1, rows1, semr1, seme1, sems1),
            (stb2, sidx2, x1v2, h1v2, w1b2, rows2, semr2, seme2, sems2))

    def _stage_set(ch, b):
        _stage(ch, st_hbm, h_hbm, x1_hbm, h1_hbm,
               b[0], b[1], b[2], b[3], b[5], b[6], b[7])

    def _process_set(b):
        _process(h_hbm, x1_hbm, h1_hbm, b[0], b[1], b[2], b[3], b[4],
                 b[5], b[6], b[7], b[8], accn, accd)

    def _drain_set(b):
        _drain_scatters(h_hbm, h1_hbm, b[0], b[1], b[4], b[5], b[8])

    _stage_set(cb0, bufs[0])
    _stage_set(cb0 + 1, bufs[1])

    def _iter(q, c):
        ca = cb0 + 3 * q
        _process_set(bufs[0])

        @pl.when(q > 0)
        def _():
            _drain_set(bufs[2])
        _stage_set(ca + 2, bufs[2])

        _process_set(bufs[1])
        _drain_set(bufs[0])

        @pl.when(q < NTRI - 1)
        def _():
            _stage_set(ca + 3, bufs[0])

        _process_set(bufs[2])
        _drain_set(bufs[1])

        @pl.when(q < NTRI - 1)
        def _():
            _stage_set(ca + 4, bufs[1])
        return c
    lax.fori_loop(0, NTRI, _iter, 0)
    _drain_set(bufs[2])

    plsc.subcore_barrier()

    for k in range(RPT // CHUNK):
        buf = (rows0, rows1, rows2)[k % 3]
        pltpu.sync_copy(accn.at[pl.ds(base + k * CHUNK, CHUNK)], buf)
        pltpu.sync_copy(buf, pn_hbm.at[cid, pl.ds(base + k * CHUNK, CHUNK)])
    pltpu.sync_copy(accn.at[pl.ds(ro, rem)], rows0.at[pl.ds(0, rem)])
    pltpu.sync_copy(rows0.at[pl.ds(0, rem)], pn_hbm.at[cid, pl.ds(ro, rem)])

    pltpu.sync_copy(accd.at[pl.ds(base, RPT)], dbuf)

    @pl.when(cid == 0)
    def _():
        pltpu.sync_copy(dbuf, pd0_hbm.at[pl.ds(base, RPT)])

    @pl.when(cid == 1)
    def _():
        pltpu.sync_copy(dbuf, pd1_hbm.at[pl.ds(base, RPT)])


_sc_hop = functools.partial(
    pl.kernel,
    out_type=[
        jax.ShapeDtypeStruct((NCORES, NPAD, D), jnp.float32),
        jax.ShapeDtypeStruct((NPAD,), jnp.float32),
        jax.ShapeDtypeStruct((NPAD,), jnp.float32),
    ],
    mesh=plsc.VectorSubcoreMesh(core_axis_name="c", subcore_axis_name="s"),
    compiler_params=pltpu.CompilerParams(needs_layout_passes=False),
    scratch_types=(
        [pltpu.VMEM((2 * CHUNK,), jnp.int32),
         pltpu.VMEM((CHUNK,), jnp.int32)] * 3
        + [pltpu.VMEM((CHUNK,), jnp.float32)] * 6
        + [pltpu.VMEM((CHUNK,), jnp.float32)] * 3
        + [pltpu.VMEM((CHUNK, D), jnp.float32)] * 3
        + [pltpu.VMEM((RPT,), jnp.float32),
           pltpu.VMEM_SHARED((NPAD, D), jnp.float32),
           pltpu.VMEM_SHARED((NPAD,), jnp.float32)]
        + [pltpu.SemaphoreType.DMA] * 9
    ),
)(_sc_hop_body)


def _pro_body(x_ref, w1_ref, b1_ref, a1_ref, a2_ref,
              x0_ref, x1_ref, xa2_ref, w2c_ref):
    x0 = lax.dot_general(x_ref[...], w1_ref[...],
                         (((1,), (1,)), ((), ()))) + b1_ref[...]
    x0 = jnp.maximum(x0, 0.0)
    x0_ref[...] = x0
    x1 = lax.dot_general(x0, a1_ref[...], (((1,), (1,)), ((), ())))
    xa2 = lax.dot_general(x0, a2_ref[...], (((1,), (1,)), ((), ())))
    x1_ref[...] = x1
    xa2_ref[...] = xa2
    v = x1 + xa2
    v = jnp.where(v > 0, v, 0.2 * v)
    w2c_ref[...] = jnp.exp(v)


def _prologue(x, w1, b1, a1, a2):
    return pl.pallas_call(
        _pro_body,
        grid=(GRID,),
        in_specs=[
            pl.BlockSpec((BS, D), lambda i: (i, 0)),
            pl.BlockSpec((D, D), lambda i: (0, 0)),
            pl.BlockSpec((1, D), lambda i: (0, 0)),
            pl.BlockSpec((HOPS, D), lambda i: (0, 0)),
            pl.BlockSpec((HOPS, D), lambda i: (0, 0)),
        ],
        out_specs=[
            pl.BlockSpec((BS, D), lambda i: (i, 0)),
            pl.BlockSpec((BS, HOPS), lambda i: (i, 0)),
            pl.BlockSpec((BS, HOPS), lambda i: (i, 0)),
            pl.BlockSpec((BS, HOPS), lambda i: (i, 0)),
        ],
        out_shape=[
            jax.ShapeDtypeStruct((N, D), jnp.float32),
            jax.ShapeDtypeStruct((N, HOPS), jnp.float32),
            jax.ShapeDtypeStruct((N, HOPS), jnp.float32),
            jax.ShapeDtypeStruct((N, HOPS), jnp.float32),
        ],
    )(x, w1, b1, a1, a2)


def _combine(pn, pd, w2, xb):
    num = pn[0] + pn[1] + w2 * xb
    dv = pd[:, 0] + pd[:, 1] + w2[:, 0]
    h = num / dv[:, None]
    return jnp.where(h > 0, h, jnp.exp(h) - 1.0)


def _fin_body(pn_ref, pd_ref, w2_ref, x_ref, a2_ref, h_ref, h1_ref):
    h = _combine(pn_ref[...], pd_ref[...], w2_ref[...], x_ref[...])
    h_ref[...] = h
    h1_ref[...] = jnp.dot(h, a2_ref[...][0])[:, None]


def _finalize(pn, pd, w2, x0, a2n):
    return pl.pallas_call(
        _fin_body,
        grid=(GRID,),
        in_specs=[
            pl.BlockSpec((NCORES, BS, D), lambda i: (0, i, 0)),
            pl.BlockSpec((BS, NCORES), lambda i: (i, 0)),
            pl.BlockSpec((BS, 1), lambda i: (i, 0)),
            pl.BlockSpec((BS, D), lambda i: (i, 0)),
            pl.BlockSpec((1, D), lambda i: (0, 0)),
        ],
        out_specs=[
            pl.BlockSpec((BS, D), lambda i: (i, 0)),
            pl.BlockSpec((BS, 1), lambda i: (i, 0)),
        ],
        out_shape=[
            jax.ShapeDtypeStruct((N, D), jnp.float32),
            jax.ShapeDtypeStruct((N, 1), jnp.float32),
        ],
    )(pn, pd, w2, x0, a2n)


def _fin_last_body(pn_ref, pd_ref, w2_ref, x_ref, w2m_ref, b2_ref, o_ref):
    h = _combine(pn_ref[...], pd_ref[...], w2_ref[...], x_ref[...])
    o_ref[...] = lax.dot_general(h, w2m_ref[...],
                                 (((1,), (1,)), ((), ()))) + b2_ref[...]


def _finalize_last(pn, pd, w2, x0, w2m, b2):
    return pl.pallas_call(
        _fin_last_body,
        grid=(GRID,),
        in_specs=[
            pl.BlockSpec((NCORES, BS, D), lambda i: (0, i, 0)),
            pl.BlockSpec((BS, NCORES), lambda i: (i, 0)),
            pl.BlockSpec((BS, 1), lambda i: (i, 0)),
            pl.BlockSpec((BS, D), lambda i: (i, 0)),
            pl.BlockSpec((D, D), lambda i: (0, 0)),
            pl.BlockSpec((1, D), lambda i: (0, 0)),
        ],
        out_specs=pl.BlockSpec((BS, D), lambda i: (i, 0)),
        out_shape=jax.ShapeDtypeStruct((N, D), jnp.float32),
    )(pn, pd, w2, x0, w2m, b2)


def kernel(x, edge_index, W1, b1, A1, A2, W2, b2):
    s = edge_index[0].astype(jnp.int32)
    t = edge_index[1].astype(jnp.int32)
    npad = EPAD - E
    ar = jnp.arange(npad, dtype=jnp.int32)
    sarr = jnp.concatenate([s, N + (ar % (NPAD - N))])
    tarr = jnp.concatenate([t, ar % N])
    stline = jnp.stack([sarr.reshape(-1, CHUNK),
                        tarr.reshape(-1, CHUNK)], axis=1).reshape(-1)

    x0, x1a, xa2, w2c = _prologue(x, W1, b1.reshape(1, D), A1, A2)
    zpad = jnp.zeros((NPAD - N,), jnp.float32)
    h = x0
    h1 = xa2[:, 0]
    out = None
    for i in range(HOPS):
        x1p = jnp.concatenate([x1a[:, i], zpad])
        h1p = jnp.concatenate([h1, zpad])
        pn, pd0, pd1 = _sc_hop(h, x1p, h1p, stline)
        pdt = jnp.stack([pd0, pd1], axis=1)
        w2r = w2c[:, i].reshape(N, 1)
        if i < HOPS - 1:
            h, h1r = _finalize(pn, pdt, w2r, x0, A2[i + 1].reshape(1, D))
            h1 = h1r[:, 0]
        else:
            out = _finalize_last(pn, pdt, w2r, x0, W2, b2.reshape(1, D))
    return out


# speedup vs baseline: 28.5981x; 1.0607x over previous
import functools

import jax
import jax.numpy as jnp
from jax import lax
from jax.experimental import pallas as pl
from jax.experimental.pallas import tpu as pltpu
from jax.experimental.pallas import tpu_sc as plsc

N = 10000
D = 128
HOPS = 10
E = 320000
NCORES = 2
NSUB = 16
NTILES = NCORES * NSUB
CHUNK = 112
NCHUNK = 93
EPT = CHUNK * NCHUNK
EPAD = NTILES * EPT
NGRP = CHUNK // 16
NTRI = NCHUNK // 3
NPAD = 10112
RPT = NPAD // NSUB
BS = 2000
GRID = N // BS


def _edge_weights(x1v, h1v, w1b):
    for g in range(NGRP):
        v = x1v[pl.ds(g * 16, 16)] + h1v[pl.ds(g * 16, 16)]
        v = jnp.where(v > 0, v, 0.2 * v)
        w1b[pl.ds(g * 16, 16)] = jnp.exp(v)


def _scale_scatter(sidx, w1b, rows, accn, accd, sems):
    def _scale(g, c2):
        for j in range(16):
            e = g * 16 + j
            wb = plsc.load_gather(w1b, [jnp.full((16,), e, jnp.int32)])
            for d in range(D // 16):
                rows[e, pl.ds(d * 16, 16)] = rows[e, pl.ds(d * 16, 16)] * wb
        return c2
    lax.fori_loop(0, NGRP, _scale, 0)
    pltpu.async_copy(rows, accn.at[sidx], sems, add=True)
    pltpu.async_copy(w1b, accd.at[sidx], sems, add=True)


def _stage_idx(ch, st_hbm, stb, semi):
    pltpu.async_copy(st_hbm.at[pl.ds(ch * 2 * CHUNK, 2 * CHUNK)], stb, semi)


def _launch(ch, st_hbm, h_hbm, x1_hbm, h1_hbm,
            stb, x1v, h1v, rows, semi, semr, seme):
    pltpu.make_async_copy(
        st_hbm.at[pl.ds(ch * 2 * CHUNK, 2 * CHUNK)], stb, semi).wait()
    tref = stb.at[pl.ds(CHUNK, CHUNK)]
    sref = stb.at[pl.ds(0, CHUNK)]
    pltpu.async_copy(h_hbm.at[tref], rows, semr)
    pltpu.async_copy(x1_hbm.at[sref], x1v, seme)
    pltpu.async_copy(h1_hbm.at[tref], h1v, seme)


def _drain_scatters(h_hbm, h1_hbm, stb, sidx, w1b, rows, sems):
    tref = stb.at[pl.ds(CHUNK, CHUNK)]
    pltpu.make_async_copy(h_hbm.at[tref], rows, sems).wait()
    pltpu.make_async_copy(h1_hbm.at[sidx], w1b, sems).wait()


def _process(h_hbm, x1_hbm, h1_hbm, stb, sidx, x1v, h1v, w1b, rows,
             semr, seme, sems, accn, accd):
    for g in range(NGRP):
        sidx[pl.ds(g * 16, 16)] = stb[pl.ds(g * 16, 16)]
    sref = stb.at[pl.ds(0, CHUNK)]
    tref = stb.at[pl.ds(CHUNK, CHUNK)]
    pltpu.make_async_copy(x1_hbm.at[sref], x1v, seme).wait()
    pltpu.make_async_copy(h1_hbm.at[tref], h1v, seme).wait()
    _edge_weights(x1v, h1v, w1b)
    pltpu.make_async_copy(h_hbm.at[tref], rows, semr).wait()
    _scale_scatter(sidx, w1b, rows, accn, accd, sems)


def _sc_hop_body(h_hbm, x1_hbm, h1_hbm, st_hbm, pn_hbm, pd0_hbm,
                 pd1_hbm, stb0, sidx0, stb1, sidx1, stb2, sidx2,
                 x1v0, h1v0, x1v1, h1v1, x1v2, h1v2, w1b0, w1b1, w1b2,
                 rows0, rows1, rows2, dbuf, accn, accd,
                 semr0, seme0, sems0, semr1, seme1, sems1,
                 semr2, seme2, sems2, semi0, semi1, semi2):
    cid = lax.axis_index("c")
    sid = lax.axis_index("s")
    wid = cid * NSUB + sid
    base = sid * RPT

    def _zrow(e, c):
        for d in range(D // 16):
            rows0[e, pl.ds(d * 16, 16)] = jnp.zeros((16,), jnp.float32)
        return c
    lax.fori_loop(0, CHUNK, _zrow, 0)
    for g in range(NGRP):
        w1b0[pl.ds(g * 16, 16)] = jnp.zeros((16,), jnp.float32)
    rem = RPT % CHUNK
    ro = base + (RPT // CHUNK) * CHUNK
    for k in range(RPT // CHUNK):
        pltpu.sync_copy(rows0, accn.at[pl.ds(base + k * CHUNK, CHUNK)])
        pltpu.sync_copy(w1b0, accd.at[pl.ds(base + k * CHUNK, CHUNK)])
    pltpu.sync_copy(rows0.at[pl.ds(0, rem)], accn.at[pl.ds(ro, rem)])
    pltpu.sync_copy(w1b0.at[pl.ds(0, rem)], accd.at[pl.ds(ro, rem)])

    plsc.subcore_barrier()

    cb0 = wid * NCHUNK
    bufs = ((stb0, sidx0, x1v0, h1v0, w1b0, rows0, semr0, seme0, sems0,
             semi0),
            (stb1, sidx1, x1v1, h1v1, w1b1, rows1, semr1, seme1, sems1,
             semi1),
            (stb2, sidx2, x1v2, h1v2, w1b2, rows2, semr2, seme2, sems2,
             semi2))

    def _idx_set(ch, b):
        _stage_idx(ch, st_hbm, b[0], b[9])

    def _launch_set(ch, b):
        _launch(ch, st_hbm, h_hbm, x1_hbm, h1_hbm,
                b[0], b[2], b[3], b[5], b[9], b[6], b[7])

    def _process_set(b):
        _process(h_hbm, x1_hbm, h1_hbm, b[0], b[1], b[2], b[3], b[4],
                 b[5], b[6], b[7], b[8], accn, accd)

    def _drain_set(b):
        _drain_scatters(h_hbm, h1_hbm, b[0], b[1], b[4], b[5], b[8])

    _idx_set(cb0, bufs[0])
    _idx_set(cb0 + 1, bufs[1])
    _idx_set(cb0 + 2, bufs[2])
    _launch_set(cb0, bufs[0])
    _launch_set(cb0 + 1, bufs[1])

    def _iter(q, c):
        ca = cb0 + 3 * q
        last = q >= NTRI - 1
        _process_set(bufs[0])

        @pl.when(q > 0)
        def _():
            _drain_set(bufs[2])
        _launch_set(ca + 2, bufs[2])

        @pl.when(jnp.logical_not(last))
        def _():
            _idx_set(ca + 3, bufs[0])

        _process_set(bufs[1])
        _drain_set(bufs[0])

        @pl.when(jnp.logical_not(last))
        def _():
            _launch_set(ca + 3, bufs[0])
            _idx_set(ca + 4, bufs[1])

        _process_set(bufs[2])
        _drain_set(bufs[1])

        @pl.when(jnp.logical_not(last))
        def _():
            _launch_set(ca + 4, bufs[1])
            _idx_set(ca + 5, bufs[2])
        return c
    lax.fori_loop(0, NTRI, _iter, 0)
    _drain_set(bufs[2])

    plsc.subcore_barrier()

    for k in range(RPT // CHUNK):
        buf = (rows0, rows1, rows2)[k % 3]
        pltpu.sync_copy(accn.at[pl.ds(base + k * CHUNK, CHUNK)], buf)
        pltpu.sync_copy(buf, pn_hbm.at[cid, pl.ds(base + k * CHUNK, CHUNK)])
    pltpu.sync_copy(accn.at[pl.ds(ro, rem)], rows0.at[pl.ds(0, rem)])
    pltpu.sync_copy(rows0.at[pl.ds(0, rem)], pn_hbm.at[cid, pl.ds(ro, rem)])

    pltpu.sync_copy(accd.at[pl.ds(base, RPT)], dbuf)

    @pl.when(cid == 0)
    def _():
        pltpu.sync_copy(dbuf, pd0_hbm.at[pl.ds(base, RPT)])

    @pl.when(cid == 1)
    def _():
        pltpu.sync_copy(dbuf, pd1_hbm.at[pl.ds(base, RPT)])


_sc_hop = functools.partial(
    pl.kernel,
    out_type=[
        jax.ShapeDtypeStruct((NCORES, NPAD, D), jnp.float32),
        jax.ShapeDtypeStruct((NPAD,), jnp.float32),
        jax.ShapeDtypeStruct((NPAD,), jnp.float32),
    ],
    mesh=plsc.VectorSubcoreMesh(core_axis_name="c", subcore_axis_name="s"),
    compiler_params=pltpu.CompilerParams(needs_layout_passes=False),
    scratch_types=(
        [pltpu.VMEM((2 * CHUNK,), jnp.int32),
         pltpu.VMEM((CHUNK,), jnp.int32)] * 3
        + [pltpu.VMEM((CHUNK,), jnp.float32)] * 6
        + [pltpu.VMEM((CHUNK,), jnp.float32)] * 3
        + [pltpu.VMEM((CHUNK, D), jnp.float32)] * 3
        + [pltpu.VMEM((RPT,), jnp.float32),
           pltpu.VMEM_SHARED((NPAD, D), jnp.float32),
           pltpu.VMEM_SHARED((NPAD,), jnp.float32)]
        + [pltpu.SemaphoreType.DMA] * 12
    ),
)(_sc_hop_body)


def _pro_body(x_ref, w1_ref, b1_ref, a1_ref, a2_ref,
              x0_ref, x1_ref, xa2_ref, w2c_ref):
    x0 = lax.dot_general(x_ref[...], w1_ref[...],
                         (((1,), (1,)), ((), ()))) + b1_ref[...]
    x0 = jnp.maximum(x0, 0.0)
    x0_ref[...] = x0
    x1 = lax.dot_general(x0, a1_ref[...], (((1,), (1,)), ((), ())))
    xa2 = lax.dot_general(x0, a2_ref[...], (((1,), (1,)), ((), ())))
    x1_ref[...] = x1
    xa2_ref[...] = xa2
    v = x1 + xa2
    v = jnp.where(v > 0, v, 0.2 * v)
    w2c_ref[...] = jnp.exp(v)


def _prologue(x, w1, b1, a1, a2):
    return pl.pallas_call(
        _pro_body,
        grid=(GRID,),
        in_specs=[
            pl.BlockSpec((BS, D), lambda i: (i, 0)),
            pl.BlockSpec((D, D), lambda i: (0, 0)),
            pl.BlockSpec((1, D), lambda i: (0, 0)),
            pl.BlockSpec((HOPS, D), lambda i: (0, 0)),
            pl.BlockSpec((HOPS, D), lambda i: (0, 0)),
        ],
        out_specs=[
            pl.BlockSpec((BS, D), lambda i: (i, 0)),
            pl.BlockSpec((BS, HOPS), lambda i: (i, 0)),
            pl.BlockSpec((BS, HOPS), lambda i: (i, 0)),
            pl.BlockSpec((BS, HOPS), lambda i: (i, 0)),
        ],
        out_shape=[
            jax.ShapeDtypeStruct((N, D), jnp.float32),
            jax.ShapeDtypeStruct((N, HOPS), jnp.float32),
            jax.ShapeDtypeStruct((N, HOPS), jnp.float32),
            jax.ShapeDtypeStruct((N, HOPS), jnp.float32),
        ],
    )(x, w1, b1, a1, a2)


def _combine(pn, pd, w2, xb):
    num = pn[0] + pn[1] + w2 * xb
    dv = pd[:, 0] + pd[:, 1] + w2[:, 0]
    h = num / dv[:, None]
    return jnp.where(h > 0, h, jnp.exp(h) - 1.0)


def _fin_body(pn_ref, pd_ref, w2_ref, x_ref, a2_ref, h_ref, h1_ref):
    h = _combine(pn_ref[...], pd_ref[...], w2_ref[...], x_ref[...])
    h_ref[...] = h
    h1_ref[...] = jnp.dot(h, a2_ref[...][0])[:, None]


def _finalize(pn, pd, w2, x0, a2n):
    return pl.pallas_call(
        _fin_body,
        grid=(GRID,),
        in_specs=[
            pl.BlockSpec((NCORES, BS, D), lambda i: (0, i, 0)),
            pl.BlockSpec((BS, NCORES), lambda i: (i, 0)),
            pl.BlockSpec((BS, 1), lambda i: (i, 0)),
            pl.BlockSpec((BS, D), lambda i: (i, 0)),
            pl.BlockSpec((1, D), lambda i: (0, 0)),
        ],
        out_specs=[
            pl.BlockSpec((BS, D), lambda i: (i, 0)),
            pl.BlockSpec((BS, 1), lambda i: (i, 0)),
        ],
        out_shape=[
            jax.ShapeDtypeStruct((N, D), jnp.float32),
            jax.ShapeDtypeStruct((N, 1), jnp.float32),
        ],
    )(pn, pd, w2, x0, a2n)


def _fin_last_body(pn_ref, pd_ref, w2_ref, x_ref, w2m_ref, b2_ref, o_ref):
    h = _combine(pn_ref[...], pd_ref[...], w2_ref[...], x_ref[...])
    o_ref[...] = lax.dot_general(h, w2m_ref[...],
                                 (((1,), (1,)), ((), ()))) + b2_ref[...]


def _finalize_last(pn, pd, w2, x0, w2m, b2):
    return pl.pallas_call(
        _fin_last_body,
        grid=(GRID,),
        in_specs=[
            pl.BlockSpec((NCORES, BS, D), lambda i: (0, i, 0)),
            pl.BlockSpec((BS, NCORES), lambda i: (i, 0)),
            pl.BlockSpec((BS, 1), lambda i: (i, 0)),
            pl.BlockSpec((BS, D), lambda i: (i, 0)),
            pl.BlockSpec((D, D), lambda i: (0, 0)),
            pl.BlockSpec((1, D), lambda i: (0, 0)),
        ],
        out_specs=pl.BlockSpec((BS, D), lambda i: (i, 0)),
        out_shape=jax.ShapeDtypeStruct((N, D), jnp.float32),
    )(pn, pd, w2, x0, w2m, b2)


def kernel(x, edge_index, W1, b1, A1, A2, W2, b2):
    s = edge_index[0].astype(jnp.int32)
    t = edge_index[1].astype(jnp.int32)
    npad = EPAD - E
    ar = jnp.arange(npad, dtype=jnp.int32)
    sarr = jnp.concatenate([s, N + (ar % (NPAD - N))])
    tarr = jnp.concatenate([t, ar % N])
    stline = jnp.stack([sarr.reshape(-1, CHUNK),
                        tarr.reshape(-1, CHUNK)], axis=1).reshape(-1)

    x0, x1a, xa2, w2c = _prologue(x, W1, b1.reshape(1, D), A1, A2)
    zpad = jnp.zeros((NPAD - N,), jnp.float32)
    h = x0
    h1 = xa2[:, 0]
    out = None
    for i in range(HOPS):
        x1p = jnp.concatenate([x1a[:, i], zpad])
        h1p = jnp.concatenate([h1, zpad])
        pn, pd0, pd1 = _sc_hop(h, x1p, h1p, stline)
        pdt = jnp.stack([pd0, pd1], axis=1)
        w2r = w2c[:, i].reshape(N, 1)
        if i < HOPS - 1:
            h, h1r = _finalize(pn, pdt, w2r, x0, A2[i + 1].reshape(1, D))
            h1 = h1r[:, 0]
        else:
            out = _finalize_last(pn, pdt, w2r, x0, W2, b2.reshape(1, D))
    return out


# speedup vs baseline: 31.6410x; 1.1064x over previous
import functools

import jax
import jax.numpy as jnp
from jax import lax
from jax.experimental import pallas as pl
from jax.experimental.pallas import tpu as pltpu
from jax.experimental.pallas import tpu_sc as plsc

N = 10000
D = 128
HOPS = 10
E = 320000
NCORES = 2
NSUB = 16
NTILES = NCORES * NSUB
CHUNK = 112
NCHUNK = 90
EPT = CHUNK * NCHUNK
EPAD = NTILES * EPT
NGRP = CHUNK // 16
NTRI = NCHUNK // 3
NPAD = 10112
RPT = NPAD // NSUB
BS = 2000
GRID = N // BS


def _edge_weights(x1v, h1v, w1b):
    for g in range(NGRP):
        v = x1v[pl.ds(g * 16, 16)] + h1v[pl.ds(g * 16, 16)]
        v = jnp.where(v > 0, v, 0.2 * v)
        w1b[pl.ds(g * 16, 16)] = jnp.exp(v)


def _scale_scatter(sidx, w1b, rows, accn, accd, sems):
    def _scale(g, c2):
        for j in range(16):
            e = g * 16 + j
            wb = plsc.load_gather(w1b, [jnp.full((16,), e, jnp.int32)])
            for d in range(D // 16):
                rows[e, pl.ds(d * 16, 16)] = rows[e, pl.ds(d * 16, 16)] * wb
        return c2
    lax.fori_loop(0, NGRP, _scale, 0)
    pltpu.async_copy(rows, accn.at[sidx], sems, add=True)
    pltpu.async_copy(w1b, accd.at[sidx], sems, add=True)


def _stage_idx(ch, st_hbm, stb, semi):
    pltpu.async_copy(st_hbm.at[pl.ds(ch * 2 * CHUNK, 2 * CHUNK)], stb, semi)


def _launch(ch, st_hbm, h_hbm, x1_hbm, h1_hbm,
            stb, x1v, h1v, rows, semi, semr, seme):
    pltpu.make_async_copy(
        st_hbm.at[pl.ds(ch * 2 * CHUNK, 2 * CHUNK)], stb, semi).wait()
    tref = stb.at[pl.ds(CHUNK, CHUNK)]
    sref = stb.at[pl.ds(0, CHUNK)]
    pltpu.async_copy(h_hbm.at[tref], rows, semr)
    pltpu.async_copy(x1_hbm.at[sref], x1v, seme)
    pltpu.async_copy(h1_hbm.at[tref], h1v, seme)


def _drain_scatters(h_hbm, h1_hbm, stb, sidx, w1b, rows, sems):
    tref = stb.at[pl.ds(CHUNK, CHUNK)]
    pltpu.make_async_copy(h_hbm.at[tref], rows, sems).wait()
    pltpu.make_async_copy(h1_hbm.at[sidx], w1b, sems).wait()


def _process(h_hbm, x1_hbm, h1_hbm, stb, sidx, x1v, h1v, w1b, rows,
             semr, seme, sems, accn, accd):
    for g in range(NGRP):
        sidx[pl.ds(g * 16, 16)] = stb[pl.ds(g * 16, 16)]
    sref = stb.at[pl.ds(0, CHUNK)]
    tref = stb.at[pl.ds(CHUNK, CHUNK)]
    pltpu.make_async_copy(x1_hbm.at[sref], x1v, seme).wait()
    pltpu.make_async_copy(h1_hbm.at[tref], h1v, seme).wait()
    _edge_weights(x1v, h1v, w1b)
    pltpu.make_async_copy(h_hbm.at[tref], rows, semr).wait()
    _scale_scatter(sidx, w1b, rows, accn, accd, sems)


def _sc_hop_body(h_hbm, x1_hbm, h1_hbm, st_hbm, pn_hbm, pd0_hbm,
                 pd1_hbm, stb0, sidx0, stb1, sidx1, stb2, sidx2,
                 x1v0, h1v0, x1v1, h1v1, x1v2, h1v2, w1b0, w1b1, w1b2,
                 rows0, rows1, rows2, dbuf, accn, accd,
                 semr0, seme0, sems0, semr1, seme1, sems1,
                 semr2, seme2, sems2, semi0, semi1, semi2):
    cid = lax.axis_index("c")
    sid = lax.axis_index("s")
    wid = cid * NSUB + sid
    base = sid * RPT

    def _zrow(e, c):
        for d in range(D // 16):
            rows0[e, pl.ds(d * 16, 16)] = jnp.zeros((16,), jnp.float32)
        return c
    lax.fori_loop(0, CHUNK, _zrow, 0)
    for g in range(NGRP):
        w1b0[pl.ds(g * 16, 16)] = jnp.zeros((16,), jnp.float32)
    rem = RPT % CHUNK
    ro = base + (RPT // CHUNK) * CHUNK
    for k in range(RPT // CHUNK):
        pltpu.sync_copy(rows0, accn.at[pl.ds(base + k * CHUNK, CHUNK)])
        pltpu.sync_copy(w1b0, accd.at[pl.ds(base + k * CHUNK, CHUNK)])
    pltpu.sync_copy(rows0.at[pl.ds(0, rem)], accn.at[pl.ds(ro, rem)])
    pltpu.sync_copy(w1b0.at[pl.ds(0, rem)], accd.at[pl.ds(ro, rem)])

    plsc.subcore_barrier()

    cb0 = wid * NCHUNK
    bufs = ((stb0, sidx0, x1v0, h1v0, w1b0, rows0, semr0, seme0, sems0,
             semi0),
            (stb1, sidx1, x1v1, h1v1, w1b1, rows1, semr1, seme1, sems1,
             semi1),
            (stb2, sidx2, x1v2, h1v2, w1b2, rows2, semr2, seme2, sems2,
             semi2))

    def _idx_set(ch, b):
        _stage_idx(ch, st_hbm, b[0], b[9])

    def _launch_set(ch, b):
        _launch(ch, st_hbm, h_hbm, x1_hbm, h1_hbm,
                b[0], b[2], b[3], b[5], b[9], b[6], b[7])

    def _process_set(b):
        _process(h_hbm, x1_hbm, h1_hbm, b[0], b[1], b[2], b[3], b[4],
                 b[5], b[6], b[7], b[8], accn, accd)

    def _drain_set(b):
        _drain_scatters(h_hbm, h1_hbm, b[0], b[1], b[4], b[5], b[8])

    _idx_set(cb0, bufs[0])
    _idx_set(cb0 + 1, bufs[1])
    _idx_set(cb0 + 2, bufs[2])
    _launch_set(cb0, bufs[0])
    _launch_set(cb0 + 1, bufs[1])

    def _iter(q, c):
        ca = cb0 + 3 * q
        last = q >= NTRI - 1
        _process_set(bufs[0])

        @pl.when(q > 0)
        def _():
            _drain_set(bufs[2])
        _launch_set(ca + 2, bufs[2])

        @pl.when(jnp.logical_not(last))
        def _():
            _idx_set(ca + 3, bufs[0])

        _process_set(bufs[1])
        _drain_set(bufs[0])

        @pl.when(jnp.logical_not(last))
        def _():
            _launch_set(ca + 3, bufs[0])
            _idx_set(ca + 4, bufs[1])

        _process_set(bufs[2])
        _drain_set(bufs[1])

        @pl.when(jnp.logical_not(last))
        def _():
            _launch_set(ca + 4, bufs[1])
            _idx_set(ca + 5, bufs[2])
        return c
    lax.fori_loop(0, NTRI, _iter, 0)
    _drain_set(bufs[2])

    plsc.subcore_barrier()

    for k in range(RPT // CHUNK):
        buf = (rows0, rows1, rows2)[k % 3]
        pltpu.sync_copy(accn.at[pl.ds(base + k * CHUNK, CHUNK)], buf)
        pltpu.sync_copy(buf, pn_hbm.at[cid, pl.ds(base + k * CHUNK, CHUNK)])
    pltpu.sync_copy(accn.at[pl.ds(ro, rem)], rows0.at[pl.ds(0, rem)])
    pltpu.sync_copy(rows0.at[pl.ds(0, rem)], pn_hbm.at[cid, pl.ds(ro, rem)])

    pltpu.sync_copy(accd.at[pl.ds(base, RPT)], dbuf)

    @pl.when(cid == 0)
    def _():
        pltpu.sync_copy(dbuf, pd0_hbm.at[pl.ds(base, RPT)])

    @pl.when(cid == 1)
    def _():
        pltpu.sync_copy(dbuf, pd1_hbm.at[pl.ds(base, RPT)])


_sc_hop = functools.partial(
    pl.kernel,
    out_type=[
        jax.ShapeDtypeStruct((NCORES, NPAD, D), jnp.float32),
        jax.ShapeDtypeStruct((NPAD,), jnp.float32),
        jax.ShapeDtypeStruct((NPAD,), jnp.float32),
    ],
    mesh=plsc.VectorSubcoreMesh(core_axis_name="c", subcore_axis_name="s"),
    compiler_params=pltpu.CompilerParams(needs_layout_passes=False),
    scratch_types=(
        [pltpu.VMEM((2 * CHUNK,), jnp.int32),
         pltpu.VMEM((CHUNK,), jnp.int32)] * 3
        + [pltpu.VMEM((CHUNK,), jnp.float32)] * 6
        + [pltpu.VMEM((CHUNK,), jnp.float32)] * 3
        + [pltpu.VMEM((CHUNK, D), jnp.float32)] * 3
        + [pltpu.VMEM((RPT,), jnp.float32),
           pltpu.VMEM_SHARED((NPAD, D), jnp.float32),
           pltpu.VMEM_SHARED((NPAD,), jnp.float32)]
        + [pltpu.SemaphoreType.DMA] * 12
    ),
)(_sc_hop_body)


def _pro_body(x_ref, w1_ref, b1_ref, a1_ref, a2_ref,
              x0_ref, x1_ref, xa2_ref, w2c_ref):
    x0 = lax.dot_general(x_ref[...], w1_ref[...],
                         (((1,), (1,)), ((), ()))) + b1_ref[...]
    x0 = jnp.maximum(x0, 0.0)
    x0_ref[...] = x0
    x1 = lax.dot_general(x0, a1_ref[...], (((1,), (1,)), ((), ())))
    xa2 = lax.dot_general(x0, a2_ref[...], (((1,), (1,)), ((), ())))
    x1_ref[...] = x1
    xa2_ref[...] = xa2
    v = x1 + xa2
    v = jnp.where(v > 0, v, 0.2 * v)
    w2c_ref[...] = jnp.exp(v)


def _prologue(x, w1, b1, a1, a2):
    return pl.pallas_call(
        _pro_body,
        grid=(GRID,),
        in_specs=[
            pl.BlockSpec((BS, D), lambda i: (i, 0)),
            pl.BlockSpec((D, D), lambda i: (0, 0)),
            pl.BlockSpec((1, D), lambda i: (0, 0)),
            pl.BlockSpec((HOPS, D), lambda i: (0, 0)),
            pl.BlockSpec((HOPS, D), lambda i: (0, 0)),
        ],
        out_specs=[
            pl.BlockSpec((BS, D), lambda i: (i, 0)),
            pl.BlockSpec((BS, HOPS), lambda i: (i, 0)),
            pl.BlockSpec((BS, HOPS), lambda i: (i, 0)),
            pl.BlockSpec((BS, HOPS), lambda i: (i, 0)),
        ],
        out_shape=[
            jax.ShapeDtypeStruct((N, D), jnp.float32),
            jax.ShapeDtypeStruct((N, HOPS), jnp.float32),
            jax.ShapeDtypeStruct((N, HOPS), jnp.float32),
            jax.ShapeDtypeStruct((N, HOPS), jnp.float32),
        ],
    )(x, w1, b1, a1, a2)


def _combine(pn, pd, w2, xb):
    num = pn[0] + pn[1] + w2 * xb
    dv = pd[:, 0] + pd[:, 1] + w2[:, 0]
    h = num / dv[:, None]
    return jnp.where(h > 0, h, jnp.exp(h) - 1.0)


def _fin_body(pn_ref, pd_ref, w2_ref, x_ref, a2_ref, h_ref, h1_ref):
    h = _combine(pn_ref[...], pd_ref[...], w2_ref[...], x_ref[...])
    h_ref[...] = h
    h1_ref[...] = jnp.dot(h, a2_ref[...][0])[:, None]


def _finalize(pn, pd, w2, x0, a2n):
    return pl.pallas_call(
        _fin_body,
        grid=(GRID,),
        in_specs=[
            pl.BlockSpec((NCORES, BS, D), lambda i: (0, i, 0)),
            pl.BlockSpec((BS, NCORES), lambda i: (i, 0)),
            pl.BlockSpec((BS, 1), lambda i: (i, 0)),
            pl.BlockSpec((BS, D), lambda i: (i, 0)),
            pl.BlockSpec((1, D), lambda i: (0, 0)),
        ],
        out_specs=[
            pl.BlockSpec((BS, D), lambda i: (i, 0)),
            pl.BlockSpec((BS, 1), lambda i: (i, 0)),
        ],
        out_shape=[
            jax.ShapeDtypeStruct((N, D), jnp.float32),
            jax.ShapeDtypeStruct((N, 1), jnp.float32),
        ],
    )(pn, pd, w2, x0, a2n)


def _fin_last_body(pn_ref, pd_ref, w2_ref, x_ref, w2m_ref, b2_ref, o_ref):
    h = _combine(pn_ref[...], pd_ref[...], w2_ref[...], x_ref[...])
    o_ref[...] = lax.dot_general(h, w2m_ref[...],
                                 (((1,), (1,)), ((), ()))) + b2_ref[...]


def _finalize_last(pn, pd, w2, x0, w2m, b2):
    return pl.pallas_call(
        _fin_last_body,
        grid=(GRID,),
        in_specs=[
            pl.BlockSpec((NCORES, BS, D), lambda i: (0, i, 0)),
            pl.BlockSpec((BS, NCORES), lambda i: (i, 0)),
            pl.BlockSpec((BS, 1), lambda i: (i, 0)),
            pl.BlockSpec((BS, D), lambda i: (i, 0)),
            pl.BlockSpec((D, D), lambda i: (0, 0)),
            pl.BlockSpec((1, D), lambda i: (0, 0)),
        ],
        out_specs=pl.BlockSpec((BS, D), lambda i: (i, 0)),
        out_shape=jax.ShapeDtypeStruct((N, D), jnp.float32),
    )(pn, pd, w2, x0, w2m, b2)


def kernel(x, edge_index, W1, b1, A1, A2, W2, b2):
    s = edge_index[0].astype(jnp.int32)
    t = edge_index[1].astype(jnp.int32)
    npad = EPAD - E
    ar = jnp.arange(npad, dtype=jnp.int32)
    sarr = jnp.concatenate([s, N + (ar % (NPAD - N))])
    tarr = jnp.concatenate([t, ar % N])
    stline = jnp.stack([sarr.reshape(-1, CHUNK),
                        tarr.reshape(-1, CHUNK)], axis=1).reshape(-1)

    x0, x1a, xa2, w2c = _prologue(x, W1, b1.reshape(1, D), A1, A2)
    zpad = jnp.zeros((NPAD - N,), jnp.float32)
    h = x0
    h1 = xa2[:, 0]
    out = None
    for i in range(HOPS):
        x1p = jnp.concatenate([x1a[:, i], zpad])
        h1p = jnp.concatenate([h1, zpad])
        pn, pd0, pd1 = _sc_hop(h, x1p, h1p, stline)
        pdt = jnp.stack([pd0, pd1], axis=1)
        w2r = w2c[:, i].reshape(N, 1)
        if i < HOPS - 1:
            h, h1r = _finalize(pn, pdt, w2r, x0, A2[i + 1].reshape(1, D))
            h1 = h1r[:, 0]
        else:
            out = _finalize_last(pn, pdt, w2r, x0, W2, b2.reshape(1, D))
    return out
